# baseline (speedup 1.0000x reference)
import functools

import jax
import jax.numpy as jnp
from jax.experimental import pallas as pl

B, DIM, N = 8, 64, 8192
T = 512
K = 4
NB = 8192


def _tc_body(src_ref, tok_ref, out_ref):
    s = src_ref[0]
    tok = tok_ref[0]
    tn = tok / jnp.sqrt(jnp.sum(tok * tok, axis=0, keepdims=True))
    sn = s / jnp.sqrt(jnp.sum(s * s, axis=0, keepdims=True))
    scores = jax.lax.dot_general(
        tn, sn, (((0,), (0,)), ((), ())),
        preferred_element_type=jnp.float32)
    m = jnp.max(scores, axis=0, keepdims=True)
    for _ in range(K - 1):
        m = jnp.max(jnp.where(scores < m, scores, -jnp.inf),
                    axis=0, keepdims=True)
    w = jnp.where(scores < m, 0.0, 1.0 / K)
    out = jax.lax.dot_general(
        tok, w, (((1,), (0,)), ((), ())),
        preferred_element_type=jnp.float32)
    out_ref[0] = out


@functools.partial(jax.jit, static_argnames=("interpret",))
def kernel(source, tokens, interpret=False):
    grid = (B, N // NB)
    return pl.pallas_call(
        _tc_body,
        grid=grid,
        in_specs=[
            pl.BlockSpec((1, DIM, NB), lambda b, nb: (b, 0, nb)),
            pl.BlockSpec((1, DIM, T), lambda b, nb: (0, 0, 0)),
        ],
        out_specs=pl.BlockSpec((1, DIM, NB), lambda b, nb: (b, 0, nb)),
        out_shape=jax.ShapeDtypeStruct((B, DIM, N), jnp.float32),
        interpret=interpret,
    )(source, tokens)


# speedup vs baseline: 1.3177x; 1.3177x over previous
import functools

import jax
import jax.numpy as jnp
from jax.experimental import pallas as pl

B, DIM, N = 8, 64, 8192
T = 512
K = 4
NB = 4096


def _tc_body(src_ref, tok_ref, out_ref):
    s = src_ref[0]
    tok = tok_ref[0]
    tn = tok / jnp.sqrt(jnp.sum(tok * tok, axis=0, keepdims=True))
    sn = s / jnp.sqrt(jnp.sum(s * s, axis=0, keepdims=True))
    scores = jax.lax.dot_general(
        tn, sn, (((0,), (0,)), ((), ())),
        preferred_element_type=jnp.float32)
    neg = jnp.full((8, NB), -jnp.inf, dtype=jnp.float32)
    r1, r2, r3, r4 = neg, neg, neg, neg
    for i in range(T // 8):
        v = scores[8 * i:8 * i + 8, :]
        t = jnp.maximum(r1, v)
        v = jnp.minimum(r1, v)
        r1 = t
        t = jnp.maximum(r2, v)
        v = jnp.minimum(r2, v)
        r2 = t
        t = jnp.maximum(r3, v)
        v = jnp.minimum(r3, v)
        r3 = t
        r4 = jnp.maximum(r4, v)
    cand = jnp.concatenate([r1, r2, r3, r4], axis=0)
    m = jnp.max(cand, axis=0, keepdims=True)
    for _ in range(K - 1):
        m = jnp.max(jnp.where(cand < m, cand, -jnp.inf),
                    axis=0, keepdims=True)
    w = jnp.where(scores < m, 0.0, 1.0 / K)
    out = jax.lax.dot_general(
        tok, w, (((1,), (0,)), ((), ())),
        preferred_element_type=jnp.float32)
    out_ref[0] = out


@functools.partial(jax.jit, static_argnames=("interpret",))
def kernel(source, tokens, interpret=False):
    grid = (B, N // NB)
    return pl.pallas_call(
        _tc_body,
        grid=grid,
        in_specs=[
            pl.BlockSpec((1, DIM, NB), lambda b, nb: (b, 0, nb)),
            pl.BlockSpec((1, DIM, T), lambda b, nb: (0, 0, 0)),
        ],
        out_specs=pl.BlockSpec((1, DIM, NB), lambda b, nb: (b, 0, nb)),
        out_shape=jax.ShapeDtypeStruct((B, DIM, N), jnp.float32),
        interpret=interpret,
    )(source, tokens)


# speedup vs baseline: 1.3372x; 1.0148x over previous
import functools

import jax
import jax.numpy as jnp
from jax.experimental import pallas as pl

B, DIM, N = 8, 64, 8192
T = 512
K = 4
NB = 4096


def _tc_body(src_ref, tok_ref, out_ref):
    s = src_ref[0]
    tok = tok_ref[0]
    tn = tok / jnp.sqrt(jnp.sum(tok * tok, axis=0, keepdims=True))
    sn = s / jnp.sqrt(jnp.sum(s * s, axis=0, keepdims=True))
    scores = jax.lax.dot_general(
        tn, sn, (((0,), (0,)), ((), ())),
        preferred_element_type=jnp.float32)
    neg = jnp.full((8, NB), -jnp.inf, dtype=jnp.float32)
    r1, r2, r3, r4 = neg, neg, neg, neg
    for i in range(T // 8):
        v = scores[8 * i:8 * i + 8, :]
        t = jnp.maximum(r1, v)
        v = jnp.minimum(r1, v)
        r1 = t
        t = jnp.maximum(r2, v)
        v = jnp.minimum(r2, v)
        r2 = t
        t = jnp.maximum(r3, v)
        v = jnp.minimum(r3, v)
        r3 = t
        r4 = jnp.maximum(r4, v)
    cand = jnp.concatenate([r1, r2, r3, r4], axis=0)
    m = jnp.max(cand, axis=0, keepdims=True)
    for _ in range(K - 1):
        m = jnp.max(jnp.where(cand < m, cand, -jnp.inf),
                    axis=0, keepdims=True)
    w = jnp.where(scores < m, 0.0, 1.0 / K).astype(jnp.bfloat16)
    out = jax.lax.dot_general(
        tok.astype(jnp.bfloat16), w, (((1,), (0,)), ((), ())),
        preferred_element_type=jnp.float32)
    out_ref[0] = out


@functools.partial(jax.jit, static_argnames=("interpret",))
def kernel(source, tokens, interpret=False):
    grid = (B, N // NB)
    return pl.pallas_call(
        _tc_body,
        grid=grid,
        in_specs=[
            pl.BlockSpec((1, DIM, NB), lambda b, nb: (b, 0, nb)),
            pl.BlockSpec((1, DIM, T), lambda b, nb: (0, 0, 0)),
        ],
        out_specs=pl.BlockSpec((1, DIM, NB), lambda b, nb: (b, 0, nb)),
        out_shape=jax.ShapeDtypeStruct((B, DIM, N), jnp.float32),
        interpret=interpret,
    )(source, tokens)
